# baseline (speedup 1.0000x reference)
import numpy as np
import jax
import jax.numpy as jnp
from jax import lax
from jax.experimental import pallas as pl
from jax.experimental.pallas import tpu as pltpu

TB = 256


def _conv1_index():
    idx = np.full((576, 784), 25, np.int32)
    for oh in range(24):
        for ow in range(24):
            for kh in range(5):
                for kw in range(5):
                    idx[oh * 24 + ow, (oh + kh) * 28 + (ow + kw)] = kh * 5 + kw
    return idx


def _conv2_index():
    idx = np.full((64, 1152), 200, np.int32)
    for oh in range(8):
        for ow in range(8):
            for ic in range(8):
                for kh in range(5):
                    for kw in range(5):
                        q = ic * 144 + (oh + kh) * 12 + (ow + kw)
                        idx[oh * 8 + ow, q] = ic * 25 + kh * 5 + kw
    return idx


_IDX1 = _conv1_index()
_IDX2 = _conv2_index()


def _lenet_kernel(x_ref, a1_ref, a2_ref, f1_ref, f2_ref, cw_ref,
                  b1_ref, b2_ref, bf_ref, bc_ref, o_ref):
    xb = x_ref[...].astype(jnp.bfloat16)

    z1 = lax.dot_general(a1_ref[...], xb, (((1,), (1,)), ((), ())),
                         preferred_element_type=jnp.float32)
    z1 = jnp.maximum(z1 + b1_ref[...], 0.0)

    z = z1.reshape(8, 24, 12, 2, TB)
    z = jnp.maximum(z[:, :, :, 0, :], z[:, :, :, 1, :])
    z = z.reshape(8, 12, 2, 12, TB)
    p1 = jnp.maximum(z[:, :, 0], z[:, :, 1])
    p1 = p1.reshape(1152, TB).astype(jnp.bfloat16)

    z2 = jnp.dot(a2_ref[...], p1, preferred_element_type=jnp.float32)
    z2 = jnp.maximum(z2 + b2_ref[...], 0.0)

    q = z2.reshape(16, 8, 4, 2, TB)
    q = jnp.maximum(q[:, :, :, 0], q[:, :, :, 1])
    q = q.reshape(16, 4, 2, 4, TB)
    flat = jnp.maximum(q[:, :, 0], q[:, :, 1]).reshape(256, TB)

    h1 = jnp.dot(f1_ref[...], flat, preferred_element_type=jnp.float32)
    h1 = jnp.maximum(h1 + bf_ref[0:64], 0.0)
    h2 = jnp.dot(f2_ref[...], h1, preferred_element_type=jnp.float32)
    h2 = jnp.maximum(h2 + bf_ref[64:96], 0.0)
    lg = lax.dot_general(h2, cw_ref[...], (((0,), (1,)), ((), ())),
                         preferred_element_type=jnp.float32)
    o_ref[...] = lg + bc_ref[...]


def kernel(x, conv1_w, conv1_b, conv2_w, conv2_b, fc1_w, fc1_b,
           fc2_w, fc2_b, cls_w, cls_b):
    b = x.shape[0]
    xf = x.reshape(b, 784)
    bp = ((b + TB - 1) // TB) * TB
    if bp != b:
        xf = jnp.pad(xf, ((0, bp - b), (0, 0)))

    w1e = jnp.concatenate([conv1_w.reshape(8, 25),
                           jnp.zeros((8, 1), conv1_w.dtype)], axis=1)
    a1 = w1e.astype(jnp.bfloat16)[:, _IDX1].reshape(4608, 784)
    w2e = jnp.concatenate([conv2_w.reshape(16, 200),
                           jnp.zeros((16, 1), conv2_w.dtype)], axis=1)
    a2 = w2e.astype(jnp.bfloat16)[:, _IDX2].reshape(1024, 1152)

    b1 = jnp.repeat(conv1_b, 576)[:, None]
    b2 = jnp.repeat(conv2_b, 64)[:, None]
    bf = jnp.zeros((128,), jnp.float32).at[0:64].set(fc1_b)
    bf = bf.at[64:96].set(fc2_b)[:, None]
    bc = cls_b[None, :]

    out = pl.pallas_call(
        _lenet_kernel,
        out_shape=jax.ShapeDtypeStruct((bp, 10), jnp.float32),
        grid=(bp // TB,),
        in_specs=[
            pl.BlockSpec((TB, 784), lambda i: (i, 0)),
            pl.BlockSpec((4608, 784), lambda i: (0, 0)),
            pl.BlockSpec((1024, 1152), lambda i: (0, 0)),
            pl.BlockSpec((64, 256), lambda i: (0, 0)),
            pl.BlockSpec((32, 64), lambda i: (0, 0)),
            pl.BlockSpec((10, 32), lambda i: (0, 0)),
            pl.BlockSpec((4608, 1), lambda i: (0, 0)),
            pl.BlockSpec((1024, 1), lambda i: (0, 0)),
            pl.BlockSpec((128, 1), lambda i: (0, 0)),
            pl.BlockSpec((1, 10), lambda i: (0, 0)),
        ],
        out_specs=pl.BlockSpec((TB, 10), lambda i: (i, 0)),
        compiler_params=pltpu.CompilerParams(
            dimension_semantics=("parallel",),
            vmem_limit_bytes=48 * 1024 * 1024,
        ),
    )(xf, a1, a2, fc1_w, fc2_w, cls_w, b1, b2, bf, bc)
    return out[:b]


# speedup vs baseline: 3.0807x; 3.0807x over previous
import jax
import jax.numpy as jnp
from jax import lax
from jax.experimental import pallas as pl
from jax.experimental.pallas import tpu as pltpu

TB = 256


def _toeplitz(w, out_len, in_len):
    lead = w.shape[:-1]
    v = jnp.pad(w, [(0, 0)] * len(lead) + [(0, in_len + 1 - w.shape[-1])])
    t = jnp.tile(v, (1,) * len(lead) + (out_len,))
    t = t[..., : out_len * in_len]
    return t.reshape(*lead, out_len, in_len)


def _conv_matrices(conv1_w, conv2_w):
    w1 = conv1_w.reshape(8, 5, 5).astype(jnp.bfloat16)
    t1 = _toeplitz(w1, 24, 28)
    t1 = _toeplitz(t1.transpose(0, 2, 3, 1), 24, 28)
    a1 = t1.transpose(0, 3, 1, 4, 2).reshape(4608, 784)

    w2 = conv2_w.astype(jnp.bfloat16)
    t2 = _toeplitz(w2, 8, 12)
    t2 = _toeplitz(t2.transpose(0, 1, 3, 4, 2), 8, 12)
    a2 = t2.transpose(0, 4, 2, 1, 5, 3).reshape(1024, 1152)
    return a1, a2


def _lenet_kernel(x_ref, a1_ref, a2_ref, f1_ref, f2_ref, cw_ref,
                  b1_ref, b2_ref, bf_ref, bc_ref, o_ref):
    xb = x_ref[...].astype(jnp.bfloat16)

    z1 = lax.dot_general(a1_ref[...], xb, (((1,), (1,)), ((), ())),
                         preferred_element_type=jnp.float32)
    z1 = jnp.maximum(z1 + b1_ref[...], 0.0)

    z = z1.reshape(8, 24, 12, 2, TB)
    z = jnp.maximum(z[:, :, :, 0, :], z[:, :, :, 1, :])
    z = z.reshape(8, 12, 2, 12, TB)
    p1 = jnp.maximum(z[:, :, 0], z[:, :, 1])
    p1 = p1.reshape(1152, TB).astype(jnp.bfloat16)

    z2 = jnp.dot(a2_ref[...], p1, preferred_element_type=jnp.float32)
    z2 = jnp.maximum(z2 + b2_ref[...], 0.0)

    q = z2.reshape(16, 8, 4, 2, TB)
    q = jnp.maximum(q[:, :, :, 0], q[:, :, :, 1])
    q = q.reshape(16, 4, 2, 4, TB)
    flat = jnp.maximum(q[:, :, 0], q[:, :, 1]).reshape(256, TB)

    h1 = jnp.dot(f1_ref[...], flat, preferred_element_type=jnp.float32)
    h1 = jnp.maximum(h1 + bf_ref[0:64], 0.0)
    h2 = jnp.dot(f2_ref[...], h1, preferred_element_type=jnp.float32)
    h2 = jnp.maximum(h2 + bf_ref[64:96], 0.0)
    lg = lax.dot_general(h2, cw_ref[...], (((0,), (1,)), ((), ())),
                         preferred_element_type=jnp.float32)
    o_ref[...] = lg + bc_ref[...]


def kernel(x, conv1_w, conv1_b, conv2_w, conv2_b, fc1_w, fc1_b,
           fc2_w, fc2_b, cls_w, cls_b):
    b = x.shape[0]
    xf = x.reshape(b, 784)
    bp = ((b + TB - 1) // TB) * TB
    if bp != b:
        xf = jnp.pad(xf, ((0, bp - b), (0, 0)))

    a1, a2 = _conv_matrices(conv1_w, conv2_w)

    b1 = jnp.repeat(conv1_b, 576)[:, None]
    b2 = jnp.repeat(conv2_b, 64)[:, None]
    bf = jnp.zeros((128,), jnp.float32).at[0:64].set(fc1_b)
    bf = bf.at[64:96].set(fc2_b)[:, None]
    bc = cls_b[None, :]

    out = pl.pallas_call(
        _lenet_kernel,
        out_shape=jax.ShapeDtypeStruct((bp, 10), jnp.float32),
        grid=(bp // TB,),
        in_specs=[
            pl.BlockSpec((TB, 784), lambda i: (i, 0)),
            pl.BlockSpec((4608, 784), lambda i: (0, 0)),
            pl.BlockSpec((1024, 1152), lambda i: (0, 0)),
            pl.BlockSpec((64, 256), lambda i: (0, 0)),
            pl.BlockSpec((32, 64), lambda i: (0, 0)),
            pl.BlockSpec((10, 32), lambda i: (0, 0)),
            pl.BlockSpec((4608, 1), lambda i: (0, 0)),
            pl.BlockSpec((1024, 1), lambda i: (0, 0)),
            pl.BlockSpec((128, 1), lambda i: (0, 0)),
            pl.BlockSpec((1, 10), lambda i: (0, 0)),
        ],
        out_specs=pl.BlockSpec((TB, 10), lambda i: (i, 0)),
        compiler_params=pltpu.CompilerParams(
            dimension_semantics=("parallel",),
            vmem_limit_bytes=48 * 1024 * 1024,
        ),
    )(xf, a1, a2, fc1_w, fc2_w, cls_w, b1, b2, bf, bc)
    return out[:b]


# speedup vs baseline: 7.8209x; 2.5387x over previous
import jax
import jax.numpy as jnp
from jax import lax
from jax.experimental import pallas as pl
from jax.experimental.pallas import tpu as pltpu

TB = 256


def _toeplitz(w, out_len, in_len):
    lead = w.shape[:-1]
    v = jnp.pad(w, [(0, 0)] * len(lead) + [(0, in_len + 1 - w.shape[-1])])
    t = jnp.tile(v, (1,) * len(lead) + (out_len,))
    t = t[..., : out_len * in_len]
    return t.reshape(*lead, out_len, in_len)


def _conv_matrices(conv1_w, conv2_w):
    w1 = conv1_w.reshape(8, 5, 5).astype(jnp.bfloat16)
    t1 = _toeplitz(w1, 24, 28)
    a = jnp.stack([jnp.pad(t1, ((0, 0), (j, 3 - j), (0, 0), (0, 0)))
                   for j in range(4)])
    a = a.reshape(2, 2, 8, 8, 12, 2, 28)
    a1q = a.transpose(5, 1, 0, 2, 4, 3, 6).reshape(768, 224)

    w2 = conv2_w.astype(jnp.bfloat16)
    t2 = _toeplitz(w2, 8, 12)
    a = jnp.stack([jnp.pad(t2, ((0, 0), (0, 0), (j, 3 - j), (0, 0), (0, 0)))
                   for j in range(4)])
    a = a.reshape(2, 2, 16, 8, 8, 4, 2, 12)
    a2q = a.transpose(6, 1, 0, 2, 5, 4, 3, 7).reshape(512, 768)
    return a1q, a2q


def _lenet_kernel(x_ref, i_ref, a1_ref, a2_ref, f1_ref, f2_ref, cw_ref,
                  b1_ref, b2_ref, bf_ref, bc_ref, o_ref, p1_ref, fl_ref):
    xb = x_ref[...].astype(jnp.bfloat16)
    xt = lax.dot_general(xb, i_ref[...], (((0,), (0,)), ((), ())),
                         preferred_element_type=jnp.float32)
    xt = xt.astype(jnp.bfloat16)

    for r in range(6):
        w = xt[112 * r:112 * r + 224, :]
        z = lax.dot_general(a1_ref[...], w, (((1,), (0,)), ((), ())),
                            preferred_element_type=jnp.float32)
        z = jnp.maximum(z[0:384], z[384:768])
        z = jnp.maximum(z[0:192], z[192:384])
        z = jnp.maximum(z + b1_ref[...], 0.0)
        p1_ref[192 * r:192 * r + 192, :] = z.astype(jnp.bfloat16)

    for q in range(2):
        w = p1_ref[384 * q:384 * q + 768, :]
        z = lax.dot_general(a2_ref[...], w, (((1,), (0,)), ((), ())),
                            preferred_element_type=jnp.float32)
        z = jnp.maximum(z[0:256], z[256:512])
        z = jnp.maximum(z[0:128], z[128:256])
        z = jnp.maximum(z + b2_ref[...], 0.0)
        fl_ref[128 * q:128 * q + 128, :] = z

    flat = fl_ref[...]
    h1 = jnp.dot(f1_ref[...], flat, preferred_element_type=jnp.float32)
    h1 = jnp.maximum(h1 + bf_ref[0:64], 0.0)
    h2 = jnp.dot(f2_ref[...], h1, preferred_element_type=jnp.float32)
    h2 = jnp.maximum(h2 + bf_ref[64:96], 0.0)
    lg = lax.dot_general(h2, cw_ref[...], (((0,), (1,)), ((), ())),
                         preferred_element_type=jnp.float32)
    o_ref[...] = lg + bc_ref[...]


def kernel(x, conv1_w, conv1_b, conv2_w, conv2_b, fc1_w, fc1_b,
           fc2_w, fc2_b, cls_w, cls_b):
    b = x.shape[0]
    xf = x.reshape(b, 784)
    bp = ((b + TB - 1) // TB) * TB
    if bp != b:
        xf = jnp.pad(xf, ((0, bp - b), (0, 0)))

    a1q, a2q = _conv_matrices(conv1_w, conv2_w)
    eye = jnp.eye(TB, dtype=jnp.bfloat16)

    b1 = jnp.tile(jnp.repeat(conv1_b, 12), 2)[:, None]
    b2 = jnp.tile(jnp.repeat(conv2_b, 4), 2)[:, None]
    bf = jnp.zeros((128,), jnp.float32).at[0:64].set(fc1_b)
    bf = bf.at[64:96].set(fc2_b)[:, None]
    bc = cls_b[None, :]
    f1p = fc1_w.reshape(64, 16, 4, 4).transpose(0, 2, 1, 3).reshape(64, 256)

    out = pl.pallas_call(
        _lenet_kernel,
        out_shape=jax.ShapeDtypeStruct((bp, 10), jnp.float32),
        grid=(bp // TB,),
        in_specs=[
            pl.BlockSpec((TB, 784), lambda i: (i, 0)),
            pl.BlockSpec((TB, TB), lambda i: (0, 0)),
            pl.BlockSpec((768, 224), lambda i: (0, 0)),
            pl.BlockSpec((512, 768), lambda i: (0, 0)),
            pl.BlockSpec((64, 256), lambda i: (0, 0)),
            pl.BlockSpec((32, 64), lambda i: (0, 0)),
            pl.BlockSpec((10, 32), lambda i: (0, 0)),
            pl.BlockSpec((192, 1), lambda i: (0, 0)),
            pl.BlockSpec((128, 1), lambda i: (0, 0)),
            pl.BlockSpec((128, 1), lambda i: (0, 0)),
            pl.BlockSpec((1, 10), lambda i: (0, 0)),
        ],
        out_specs=pl.BlockSpec((TB, 10), lambda i: (i, 0)),
        scratch_shapes=[
            pltpu.VMEM((1152, TB), jnp.bfloat16),
            pltpu.VMEM((256, TB), jnp.float32),
        ],
        compiler_params=pltpu.CompilerParams(
            dimension_semantics=("parallel",),
            vmem_limit_bytes=32 * 1024 * 1024,
        ),
    )(xf, eye, a1q, a2q, f1p, fc2_w, cls_w, b1, b2, bf, bc)
    return out[:b]


# speedup vs baseline: 9.1108x; 1.1649x over previous
import jax
import jax.numpy as jnp
from jax import lax
from jax.experimental import pallas as pl
from jax.experimental.pallas import tpu as pltpu

TB = 256


def _toeplitz(w, out_len, in_len):
    lead = w.shape[:-1]
    v = jnp.pad(w, [(0, 0)] * len(lead) + [(0, in_len + 1 - w.shape[-1])])
    t = jnp.tile(v, (1,) * len(lead) + (out_len,))
    t = t[..., : out_len * in_len]
    return t.reshape(*lead, out_len, in_len)


def _conv_matrices(conv1_w, conv2_w):
    w1 = conv1_w.reshape(8, 5, 5).astype(jnp.bfloat16)
    t1 = _toeplitz(w1, 24, 28)
    a = jnp.stack([jnp.pad(t1, ((0, 0), (j, 3 - j), (0, 0), (0, 0)))
                   for j in range(4)])
    a = a.reshape(2, 2, 8, 8, 12, 2, 28)
    a1q = a.transpose(5, 1, 0, 2, 4, 3, 6).reshape(768, 224)

    w2 = conv2_w.astype(jnp.bfloat16)
    t2 = _toeplitz(w2, 8, 12)
    a = jnp.stack([jnp.pad(t2, ((0, 0), (0, 0), (j, 3 - j), (0, 0), (0, 0)))
                   for j in range(4)])
    a = a.reshape(2, 2, 16, 8, 8, 4, 2, 12)
    a2q = a.transpose(6, 1, 0, 2, 5, 4, 3, 7).reshape(512, 768)
    return a1q, a2q


def _lenet_kernel(x_ref, a1_ref, a2_ref, f1_ref, f2_ref, cw_ref,
                  b1_ref, b2_ref, bf_ref, bc_ref, o_ref, p1_ref, fl_ref):
    xt = x_ref[...].astype(jnp.bfloat16)

    for r in range(6):
        w = xt[112 * r:112 * r + 224, :]
        z = lax.dot_general(a1_ref[...], w, (((1,), (0,)), ((), ())),
                            preferred_element_type=jnp.float32)
        z = jnp.maximum(z[0:384], z[384:768])
        z = jnp.maximum(z[0:192], z[192:384])
        z = jnp.maximum(z + b1_ref[...], 0.0)
        p1_ref[192 * r:192 * r + 192, :] = z.astype(jnp.bfloat16)

    for q in range(2):
        w = p1_ref[384 * q:384 * q + 768, :]
        z = lax.dot_general(a2_ref[...], w, (((1,), (0,)), ((), ())),
                            preferred_element_type=jnp.float32)
        z = jnp.maximum(z[0:256], z[256:512])
        z = jnp.maximum(z[0:128], z[128:256])
        z = jnp.maximum(z + b2_ref[...], 0.0)
        fl_ref[128 * q:128 * q + 128, :] = z

    flat = fl_ref[...]
    h1 = jnp.dot(f1_ref[...], flat, preferred_element_type=jnp.float32)
    h1 = jnp.maximum(h1 + bf_ref[0:64], 0.0)
    h2 = jnp.dot(f2_ref[...], h1, preferred_element_type=jnp.float32)
    h2 = jnp.maximum(h2 + bf_ref[64:96], 0.0)
    lg = lax.dot_general(h2, cw_ref[...], (((0,), (1,)), ((), ())),
                         preferred_element_type=jnp.float32)
    o_ref[...] = lg + bc_ref[...]


def kernel(x, conv1_w, conv1_b, conv2_w, conv2_b, fc1_w, fc1_b,
           fc2_w, fc2_b, cls_w, cls_b):
    b = x.shape[0]
    bp = ((b + TB - 1) // TB) * TB
    xf = x.reshape(b, 784)
    if bp != b:
        xf = jnp.pad(xf, ((0, bp - b), (0, 0)))
    xt = xf.T

    a1q, a2q = _conv_matrices(conv1_w, conv2_w)

    b1 = jnp.tile(jnp.repeat(conv1_b, 12), 2)[:, None]
    b2 = jnp.tile(jnp.repeat(conv2_b, 4), 2)[:, None]
    bf = jnp.zeros((128,), jnp.float32).at[0:64].set(fc1_b)
    bf = bf.at[64:96].set(fc2_b)[:, None]
    bc = cls_b[None, :]
    f1p = fc1_w.reshape(64, 16, 4, 4).transpose(0, 2, 1, 3).reshape(64, 256)

    out = pl.pallas_call(
        _lenet_kernel,
        out_shape=jax.ShapeDtypeStruct((bp, 10), jnp.float32),
        grid=(bp // TB,),
        in_specs=[
            pl.BlockSpec((784, TB), lambda i: (0, i)),
            pl.BlockSpec((768, 224), lambda i: (0, 0)),
            pl.BlockSpec((512, 768), lambda i: (0, 0)),
            pl.BlockSpec((64, 256), lambda i: (0, 0)),
            pl.BlockSpec((32, 64), lambda i: (0, 0)),
            pl.BlockSpec((10, 32), lambda i: (0, 0)),
            pl.BlockSpec((192, 1), lambda i: (0, 0)),
            pl.BlockSpec((128, 1), lambda i: (0, 0)),
            pl.BlockSpec((128, 1), lambda i: (0, 0)),
            pl.BlockSpec((1, 10), lambda i: (0, 0)),
        ],
        out_specs=pl.BlockSpec((TB, 10), lambda i: (i, 0)),
        scratch_shapes=[
            pltpu.VMEM((1152, TB), jnp.bfloat16),
            pltpu.VMEM((256, TB), jnp.float32),
        ],
        compiler_params=pltpu.CompilerParams(
            dimension_semantics=("parallel",),
            vmem_limit_bytes=32 * 1024 * 1024,
        ),
    )(xt, a1q, a2q, f1p, fc2_w, cls_w, b1, b2, bf, bc)
    return out[:b]
